# baseline (speedup 1.0000x reference)
import functools

import jax
import jax.numpy as jnp
from jax import lax
from jax.experimental import pallas as pl
from jax.experimental.pallas import tpu as pltpu
from jax.experimental.pallas import tpu_sc as plsc

D_FEAT = 128
LANES = 16
N_CORES = 2
N_SUBCORES = 16
N_WORKERS = N_CORES * N_SUBCORES
CHUNK = 128
GROUPS = CHUNK // LANES
VPF = D_FEAT // LANES
NBUF = 2


def _make_kernel(e_pad):
  ew = e_pad // N_WORKERS
  n_chunks = ew // CHUNK
  assert n_chunks % NBUF == 0
  mesh = plsc.VectorSubcoreMesh(core_axis_name="c", subcore_axis_name="s")

  @functools.partial(
      pl.kernel,
      mesh=mesh,
      compiler_params=pltpu.CompilerParams(needs_layout_passes=False),
      out_type=jax.ShapeDtypeStruct((e_pad,), jnp.float32),
      scratch_types=[
          pltpu.VMEM((ew,), jnp.int32),
          pltpu.VMEM((ew,), jnp.int32),
      ] + [pltpu.VMEM((CHUNK, D_FEAT), jnp.float32)] * (2 * NBUF) + [
          pltpu.VMEM((CHUNK,), jnp.float32),
      ] + [pltpu.SemaphoreType.DMA] * (2 * NBUF),
  )
  def score_kernel(h_hbm, src_hbm, dst_hbm, out_hbm,
                   idx_all_u, idx_all_v, *rest):
    rows_u = rest[:NBUF]
    rows_v = rest[NBUF:2 * NBUF]
    out_v = rest[2 * NBUF]
    sems = rest[2 * NBUF + 1:]
    wid = lax.axis_index("s") * N_CORES + lax.axis_index("c")
    base = wid * ew
    lane = lax.iota(jnp.int32, LANES)
    sem_u = sems[:NBUF]
    sem_v = sems[NBUF:]

    pltpu.sync_copy(src_hbm.at[pl.ds(base, ew)], idx_all_u)
    pltpu.sync_copy(dst_hbm.at[pl.ds(base, ew)], idx_all_v)

    def fire(ch, b):
      iu = idx_all_u.at[pl.ds(ch * CHUNK, CHUNK)]
      iv = idx_all_v.at[pl.ds(ch * CHUNK, CHUNK)]
      pltpu.async_copy(h_hbm.at[iu], rows_u[b], sem_u[b])
      pltpu.async_copy(h_hbm.at[iv], rows_v[b], sem_v[b])

    def wait(ch, b):
      iu = idx_all_u.at[pl.ds(ch * CHUNK, CHUNK)]
      iv = idx_all_v.at[pl.ds(ch * CHUNK, CHUNK)]
      pltpu.make_async_copy(h_hbm.at[iu], rows_u[b], sem_u[b]).wait()
      pltpu.make_async_copy(h_hbm.at[iv], rows_v[b], sem_v[b]).wait()

    for b in range(NBUF):
      fire(b, b)

    def loop_body(j, carry):
      for b in range(NBUF):
        ch = NBUF * j + b
        wait(ch, b)

        def group_body(g, carry2, b=b):
          acc = jnp.zeros((LANES,), jnp.float32)
          for k in range(LANES):
            e = g * LANES + k
            ms = [rows_u[b][e, pl.ds(i * LANES, LANES)]
                  * rows_v[b][e, pl.ds(i * LANES, LANES)]
                  for i in range(VPF)]
            m = ((ms[0] + ms[1]) + (ms[2] + ms[3])) + (
                (ms[4] + ms[5]) + (ms[6] + ms[7]))
            acc = jnp.where(lane == k, jnp.sum(m), acc)
          out_v[pl.ds(g * LANES, LANES)] = acc
          return carry2

        lax.fori_loop(0, GROUPS, group_body, 0)
        pltpu.sync_copy(out_v, out_hbm.at[pl.ds(base + ch * CHUNK, CHUNK)])
        fire(jnp.minimum(ch + NBUF, n_chunks - 1), b)
      return carry

    lax.fori_loop(0, n_chunks // NBUF, loop_body, 0)
    for b in range(NBUF):
      wait(0, b)

  return score_kernel


def kernel(h, edge_index):
  e = edge_index.shape[1]
  epc = N_WORKERS * CHUNK * NBUF
  e_pad = ((e + epc - 1) // epc) * epc
  src = edge_index[0].astype(jnp.int32)
  dst = edge_index[1].astype(jnp.int32)
  if e_pad != e:
    src = jnp.pad(src, (0, e_pad - e))
    dst = jnp.pad(dst, (0, e_pad - e))
  out = _make_kernel(e_pad)(h, src, dst)
  return out[:e, None]


# speedup vs baseline: 2.0070x; 2.0070x over previous
import functools

import jax
import jax.numpy as jnp
from jax import lax
from jax.experimental import pallas as pl
from jax.experimental.pallas import tpu as pltpu
from jax.experimental.pallas import tpu_sc as plsc

D_FEAT = 128
LANES = 16
N_CORES = 2
N_SUBCORES = 16
N_WORKERS = N_CORES * N_SUBCORES
CHUNK = 64
GROUPS = CHUNK // LANES
VPF = D_FEAT // LANES
NBUF = 2


def _make_kernel(e_pad, n_pad):
  ew = e_pad // N_WORKERS
  n_chunks = ew // CHUNK
  assert n_chunks % NBUF == 0
  assert n_pad % (8 * N_SUBCORES) == 0
  rows_per_sub = n_pad // N_SUBCORES
  mesh = plsc.VectorSubcoreMesh(core_axis_name="c", subcore_axis_name="s")

  @functools.partial(
      pl.kernel,
      mesh=mesh,
      compiler_params=pltpu.CompilerParams(needs_layout_passes=False),
      out_type=jax.ShapeDtypeStruct((e_pad,), jnp.float32),
      scratch_types=[
          pltpu.VMEM_SHARED((n_pad, D_FEAT), jnp.float32),
      ] + [pltpu.VMEM((2 * CHUNK,), jnp.int32)] * NBUF
        + [pltpu.VMEM((CHUNK, D_FEAT), jnp.float32)] * (2 * NBUF) + [
          pltpu.VMEM((CHUNK,), jnp.float32),
      ] + [pltpu.SemaphoreType.DMA] * (2 * NBUF),
  )
  def score_kernel(h_hbm, idx_hbm, out_hbm, h_sh, *rest):
    idx_bufs = rest[:NBUF]
    rows_u = rest[NBUF:2 * NBUF]
    rows_v = rest[2 * NBUF:3 * NBUF]
    out_v = rest[3 * NBUF]
    sems = rest[3 * NBUF + 1:]
    sem_u = sems[:NBUF]
    sem_v = sems[NBUF:]

    cid = lax.axis_index("c")
    sid = lax.axis_index("s")
    wid = sid * N_CORES + cid
    base = wid * ew
    chunk0 = wid * n_chunks
    lane = lax.iota(jnp.int32, LANES)

    pltpu.sync_copy(h_hbm.at[pl.ds(sid * rows_per_sub, rows_per_sub)],
                    h_sh.at[pl.ds(sid * rows_per_sub, rows_per_sub)])
    plsc.subcore_barrier()

    def load_idx(ch, b):
      off = (chunk0 + ch) * (2 * CHUNK)
      pltpu.sync_copy(idx_hbm.at[pl.ds(off, 2 * CHUNK)], idx_bufs[b])

    def fire(b):
      iu = idx_bufs[b].at[pl.ds(0, CHUNK)]
      iv = idx_bufs[b].at[pl.ds(CHUNK, CHUNK)]
      pltpu.async_copy(h_sh.at[iu], rows_u[b], sem_u[b])
      pltpu.async_copy(h_sh.at[iv], rows_v[b], sem_v[b])

    def wait_gather(b):
      iu = idx_bufs[b].at[pl.ds(0, CHUNK)]
      iv = idx_bufs[b].at[pl.ds(CHUNK, CHUNK)]
      pltpu.make_async_copy(h_sh.at[iu], rows_u[b], sem_u[b]).wait()
      pltpu.make_async_copy(h_sh.at[iv], rows_v[b], sem_v[b]).wait()

    for b in range(NBUF):
      load_idx(b, b)
      fire(b)

    def loop_body(j, carry):
      for b in range(NBUF):
        ch = NBUF * j + b
        wait_gather(b)

        def group_body(g, carry2, b=b):
          acc = jnp.zeros((LANES,), jnp.float32)
          for k in range(LANES):
            e = g * LANES + k
            ms = [rows_u[b][e, pl.ds(i * LANES, LANES)]
                  * rows_v[b][e, pl.ds(i * LANES, LANES)]
                  for i in range(VPF)]
            m = ((ms[0] + ms[1]) + (ms[2] + ms[3])) + (
                (ms[4] + ms[5]) + (ms[6] + ms[7]))
            acc = jnp.where(lane == k, jnp.sum(m), acc)
          out_v[pl.ds(g * LANES, LANES)] = acc
          return carry2

        lax.fori_loop(0, GROUPS, group_body, 0)
        pltpu.sync_copy(out_v, out_hbm.at[pl.ds(base + ch * CHUNK, CHUNK)])
        load_idx(jnp.minimum(ch + NBUF, n_chunks - 1), b)
        fire(b)
      return carry

    lax.fori_loop(0, n_chunks // NBUF, loop_body, 0)
    for b in range(NBUF):
      wait_gather(b)

  return score_kernel


def kernel(h, edge_index):
  e = edge_index.shape[1]
  epc = N_WORKERS * CHUNK * NBUF
  e_pad = ((e + epc - 1) // epc) * epc
  src = edge_index[0].astype(jnp.int32)
  dst = edge_index[1].astype(jnp.int32)
  if e_pad != e:
    src = jnp.pad(src, (0, e_pad - e))
    dst = jnp.pad(dst, (0, e_pad - e))
  idx = jnp.stack([src.reshape(-1, CHUNK), dst.reshape(-1, CHUNK)],
                  axis=1).reshape(-1)
  npc = 8 * N_SUBCORES
  n_pad = ((h.shape[0] + npc - 1) // npc) * npc
  if n_pad != h.shape[0]:
    h = jnp.pad(h, ((0, n_pad - h.shape[0]), (0, 0)))
  out = _make_kernel(e_pad, n_pad)(h, idx)
  return out[:e, None]


# speedup vs baseline: 2.1193x; 1.0560x over previous
import functools

import jax
import jax.numpy as jnp
from jax import lax
from jax.experimental import pallas as pl
from jax.experimental.pallas import tpu as pltpu
from jax.experimental.pallas import tpu_sc as plsc

D_FEAT = 128
LANES = 16
N_CORES = 2
N_SUBCORES = 16
N_WORKERS = N_CORES * N_SUBCORES
CHUNK = 64
GROUPS = CHUNK // LANES
VPF = D_FEAT // LANES
NBUF = 2


def _make_kernel(e_pad, n_pad):
  ew = e_pad // N_WORKERS
  n_chunks = ew // CHUNK
  assert n_chunks % NBUF == 0
  assert n_pad % (8 * N_SUBCORES) == 0
  rows_per_sub = n_pad // N_SUBCORES
  mesh = plsc.VectorSubcoreMesh(core_axis_name="c", subcore_axis_name="s")

  @functools.partial(
      pl.kernel,
      mesh=mesh,
      compiler_params=pltpu.CompilerParams(needs_layout_passes=False),
      out_type=jax.ShapeDtypeStruct((e_pad,), jnp.float32),
      scratch_types=[
          pltpu.VMEM_SHARED((n_pad, D_FEAT), jnp.float32),
      ] + [pltpu.VMEM((2 * CHUNK,), jnp.int32)] * NBUF
        + [pltpu.VMEM((CHUNK, D_FEAT), jnp.float32)] * (2 * NBUF) + [
          pltpu.VMEM((CHUNK,), jnp.float32),
      ] + [pltpu.SemaphoreType.DMA] * (2 * NBUF),
  )
  def score_kernel(h_hbm, idx_hbm, out_hbm, h_sh, *rest):
    idx_bufs = rest[:NBUF]
    rows_u = rest[NBUF:2 * NBUF]
    rows_v = rest[2 * NBUF:3 * NBUF]
    out_v = rest[3 * NBUF]
    sems = rest[3 * NBUF + 1:]
    sem_u = sems[:NBUF]
    sem_v = sems[NBUF:]

    cid = lax.axis_index("c")
    sid = lax.axis_index("s")
    wid = sid * N_CORES + cid
    base = wid * ew
    chunk0 = wid * n_chunks
    lane = lax.iota(jnp.int32, LANES)
    rots = [jnp.bitwise_and(lane + r, LANES - 1) for r in (8, 4, 2, 1)]
    places = [jnp.bitwise_and(lane - k, LANES - 1) for k in range(LANES)]

    def rot(x, perm):
      return x.at[perm].get(mode="promise_in_bounds")

    pltpu.sync_copy(h_hbm.at[pl.ds(sid * rows_per_sub, rows_per_sub)],
                    h_sh.at[pl.ds(sid * rows_per_sub, rows_per_sub)])
    plsc.subcore_barrier()

    def load_idx(ch, b):
      off = (chunk0 + ch) * (2 * CHUNK)
      pltpu.sync_copy(idx_hbm.at[pl.ds(off, 2 * CHUNK)], idx_bufs[b])

    def fire(b):
      iu = idx_bufs[b].at[pl.ds(0, CHUNK)]
      iv = idx_bufs[b].at[pl.ds(CHUNK, CHUNK)]
      pltpu.async_copy(h_sh.at[iu], rows_u[b], sem_u[b])
      pltpu.async_copy(h_sh.at[iv], rows_v[b], sem_v[b])

    def wait_gather(b):
      iu = idx_bufs[b].at[pl.ds(0, CHUNK)]
      iv = idx_bufs[b].at[pl.ds(CHUNK, CHUNK)]
      pltpu.make_async_copy(h_sh.at[iu], rows_u[b], sem_u[b]).wait()
      pltpu.make_async_copy(h_sh.at[iv], rows_v[b], sem_v[b]).wait()

    for b in range(NBUF):
      load_idx(b, b)
      fire(b)

    def loop_body(j, carry):
      for b in range(NBUF):
        ch = NBUF * j + b
        wait_gather(b)

        def group_body(g, carry2, b=b):
          acc = jnp.zeros((LANES,), jnp.float32)
          for k in range(LANES):
            e = g * LANES + k
            ms = [rows_u[b][e, pl.ds(i * LANES, LANES)]
                  * rows_v[b][e, pl.ds(i * LANES, LANES)]
                  for i in range(VPF)]
            m = ((ms[0] + ms[1]) + (ms[2] + ms[3])) + (
                (ms[4] + ms[5]) + (ms[6] + ms[7]))
            for p in rots:
              m = m + rot(m, p)
            t = m if k == 0 else rot(m, places[k])
            acc = jnp.where(lane == k, t, acc)
          out_v[pl.ds(g * LANES, LANES)] = acc
          return carry2

        lax.fori_loop(0, GROUPS, group_body, 0)
        pltpu.sync_copy(out_v, out_hbm.at[pl.ds(base + ch * CHUNK, CHUNK)])
        load_idx(jnp.minimum(ch + NBUF, n_chunks - 1), b)
        fire(b)
      return carry

    lax.fori_loop(0, n_chunks // NBUF, loop_body, 0)
    for b in range(NBUF):
      wait_gather(b)

  return score_kernel


def kernel(h, edge_index):
  e = edge_index.shape[1]
  epc = N_WORKERS * CHUNK * NBUF
  e_pad = ((e + epc - 1) // epc) * epc
  src = edge_index[0].astype(jnp.int32)
  dst = edge_index[1].astype(jnp.int32)
  if e_pad != e:
    src = jnp.pad(src, (0, e_pad - e))
    dst = jnp.pad(dst, (0, e_pad - e))
  idx = jnp.stack([src.reshape(-1, CHUNK), dst.reshape(-1, CHUNK)],
                  axis=1).reshape(-1)
  npc = 8 * N_SUBCORES
  n_pad = ((h.shape[0] + npc - 1) // npc) * npc
  if n_pad != h.shape[0]:
    h = jnp.pad(h, ((0, n_pad - h.shape[0]), (0, 0)))
  out = _make_kernel(e_pad, n_pad)(h, idx)
  return out[:e, None]


# speedup vs baseline: 5.3353x; 2.5175x over previous
import functools

import jax
import jax.numpy as jnp
from jax import lax
from jax.experimental import pallas as pl
from jax.experimental.pallas import tpu as pltpu
from jax.experimental.pallas import tpu_sc as plsc

D_FEAT = 128
LANES = 16
N_CORES = 2
N_SUBCORES = 16
N_WORKERS = N_CORES * N_SUBCORES
CHUNK = 128
GROUPS = CHUNK // LANES
D_WORDS = D_FEAT // 2
WPF = D_WORDS // LANES
NBUF = 2


def _make_kernel(e_pad, n_pad):
  ew = e_pad // N_WORKERS
  n_chunks = ew // CHUNK
  assert n_chunks % NBUF == 0
  assert n_pad % (8 * N_SUBCORES) == 0
  rows_per_sub = n_pad // N_SUBCORES
  mesh = plsc.VectorSubcoreMesh(core_axis_name="c", subcore_axis_name="s")

  @functools.partial(
      pl.kernel,
      mesh=mesh,
      compiler_params=pltpu.CompilerParams(needs_layout_passes=False,
                                           use_tc_tiling_on_sc=False),
      out_type=jax.ShapeDtypeStruct((e_pad,), jnp.float32),
      scratch_types=[
          pltpu.VMEM_SHARED((n_pad, D_WORDS), jnp.float32),
      ] + [pltpu.VMEM((2 * CHUNK,), jnp.int32)] * NBUF
        + [pltpu.VMEM((CHUNK, D_WORDS), jnp.float32)] * (2 * NBUF) + [
          pltpu.VMEM((CHUNK,), jnp.float32),
      ] + [pltpu.SemaphoreType.DMA] * (2 * NBUF),
  )
  def score_kernel(h_hbm, idx_hbm, out_hbm, h_sh, *rest):
    idx_bufs = rest[:NBUF]
    rows_u = rest[NBUF:2 * NBUF]
    rows_v = rest[2 * NBUF:3 * NBUF]
    out_v = rest[3 * NBUF]
    sems = rest[3 * NBUF + 1:]
    sem_u = sems[:NBUF]
    sem_v = sems[NBUF:]

    cid = lax.axis_index("c")
    sid = lax.axis_index("s")
    wid = sid * N_CORES + cid
    base = wid * ew
    chunk0 = wid * n_chunks
    lane = lax.iota(jnp.int32, LANES)
    rots = [jnp.bitwise_and(lane + r, LANES - 1) for r in (8, 4, 2, 1)]
    places = [jnp.bitwise_and(lane - k, LANES - 1) for k in range(LANES)]

    def rot(x, perm):
      return x.at[perm].get(mode="promise_in_bounds")

    pltpu.sync_copy(h_hbm.at[pl.ds(sid * rows_per_sub, rows_per_sub)],
                    h_sh.at[pl.ds(sid * rows_per_sub, rows_per_sub)])
    plsc.subcore_barrier()

    def load_idx(ch, b):
      off = (chunk0 + ch) * (2 * CHUNK)
      pltpu.sync_copy(idx_hbm.at[pl.ds(off, 2 * CHUNK)], idx_bufs[b])

    def fire(b):
      iu = idx_bufs[b].at[pl.ds(0, CHUNK)]
      iv = idx_bufs[b].at[pl.ds(CHUNK, CHUNK)]
      pltpu.async_copy(h_sh.at[iu], rows_u[b], sem_u[b])
      pltpu.async_copy(h_sh.at[iv], rows_v[b], sem_v[b])

    def wait_gather(b):
      iu = idx_bufs[b].at[pl.ds(0, CHUNK)]
      iv = idx_bufs[b].at[pl.ds(CHUNK, CHUNK)]
      pltpu.make_async_copy(h_sh.at[iu], rows_u[b], sem_u[b]).wait()
      pltpu.make_async_copy(h_sh.at[iv], rows_v[b], sem_v[b]).wait()

    for b in range(NBUF):
      load_idx(b, b)
      fire(b)

    def loop_body(j, carry):
      for b in range(NBUF):
        ch = NBUF * j + b
        wait_gather(b)

        def group_body(g, carry2, b=b):
          acc = jnp.zeros((LANES,), jnp.float32)
          for k in range(LANES):
            e = g * LANES + k
            ps = []
            for i in range(WPF):
              uw = plsc.bitcast(rows_u[b][e, pl.ds(i * LANES, LANES)],
                                jnp.bfloat16)
              vw = plsc.bitcast(rows_v[b][e, pl.ds(i * LANES, LANES)],
                                jnp.bfloat16)
              lo, hi = plsc.unpack(uw * vw, format=plsc.PackFormat.INTERLEAVED)
              ps.append(lo + hi)
            m = (ps[0] + ps[1]) + (ps[2] + ps[3])
            for p in rots:
              m = m + rot(m, p)
            t = m if k == 0 else rot(m, places[k])
            acc = jnp.where(lane == k, t, acc)
          out_v[pl.ds(g * LANES, LANES)] = acc
          return carry2

        lax.fori_loop(0, GROUPS, group_body, 0)
        pltpu.sync_copy(out_v, out_hbm.at[pl.ds(base + ch * CHUNK, CHUNK)])
        load_idx(jnp.minimum(ch + NBUF, n_chunks - 1), b)
        fire(b)
      return carry

    lax.fori_loop(0, n_chunks // NBUF, loop_body, 0)
    for b in range(NBUF):
      wait_gather(b)

  return score_kernel


def kernel(h, edge_index):
  e = edge_index.shape[1]
  epc = N_WORKERS * CHUNK * NBUF
  e_pad = ((e + epc - 1) // epc) * epc
  src = edge_index[0].astype(jnp.int32)
  dst = edge_index[1].astype(jnp.int32)
  if e_pad != e:
    src = jnp.pad(src, (0, e_pad - e))
    dst = jnp.pad(dst, (0, e_pad - e))
  idx = jnp.stack([src.reshape(-1, CHUNK), dst.reshape(-1, CHUNK)],
                  axis=1).reshape(-1)
  npc = 8 * N_SUBCORES
  n_pad = ((h.shape[0] + npc - 1) // npc) * npc
  if n_pad != h.shape[0]:
    h = jnp.pad(h, ((0, n_pad - h.shape[0]), (0, 0)))
  hw = jax.lax.bitcast_convert_type(
      h.astype(jnp.bfloat16).reshape(n_pad, D_FEAT // 2, 2), jnp.float32)
  out = _make_kernel(e_pad, n_pad)(hw, idx)
  return out[:e, None]


# speedup vs baseline: 6.6037x; 1.2377x over previous
import functools

import jax
import jax.numpy as jnp
from jax import lax
from jax.experimental import pallas as pl
from jax.experimental.pallas import tpu as pltpu
from jax.experimental.pallas import tpu_sc as plsc

D_FEAT = 128
LANES = 16
N_CORES = 2
N_SUBCORES = 16
N_WORKERS = N_CORES * N_SUBCORES
CHUNK = 128
GROUPS = CHUNK // LANES
D_WORDS = D_FEAT // 2
WPF = D_WORDS // LANES
NBUF = 2


def _make_kernel(e_pad, n_pad):
  ew = e_pad // N_WORKERS
  n_chunks = ew // CHUNK
  assert n_chunks % NBUF == 0
  assert n_pad % (8 * N_SUBCORES) == 0
  rows_per_sub = n_pad // N_SUBCORES
  mesh = plsc.VectorSubcoreMesh(core_axis_name="c", subcore_axis_name="s")

  @functools.partial(
      pl.kernel,
      mesh=mesh,
      compiler_params=pltpu.CompilerParams(needs_layout_passes=False,
                                           use_tc_tiling_on_sc=False),
      out_type=jax.ShapeDtypeStruct((e_pad,), jnp.float32),
      scratch_types=[
          pltpu.VMEM_SHARED((n_pad, D_WORDS), jnp.float32),
          pltpu.VMEM((2 * ew,), jnp.int32),
          pltpu.VMEM((ew,), jnp.float32),
      ] + [pltpu.VMEM((CHUNK, D_WORDS), jnp.float32)] * (2 * NBUF)
        + [pltpu.SemaphoreType.DMA] * (2 * NBUF),
  )
  def score_kernel(h_hbm, idx_hbm, out_hbm, h_sh, idx_all, out_all, *rest):
    rows_u = rest[:NBUF]
    rows_v = rest[NBUF:2 * NBUF]
    sems = rest[2 * NBUF:]
    sem_u = sems[:NBUF]
    sem_v = sems[NBUF:]

    cid = lax.axis_index("c")
    sid = lax.axis_index("s")
    wid = sid * N_CORES + cid
    base = wid * ew
    chunk0 = wid * n_chunks
    lane = lax.iota(jnp.int32, LANES)
    rots = [jnp.bitwise_and(lane + r, LANES - 1) for r in (8, 4, 2, 1)]
    places = [jnp.bitwise_and(lane - k, LANES - 1) for k in range(LANES)]

    def rot(x, perm):
      return x.at[perm].get(mode="promise_in_bounds")

    pltpu.sync_copy(h_hbm.at[pl.ds(sid * rows_per_sub, rows_per_sub)],
                    h_sh.at[pl.ds(sid * rows_per_sub, rows_per_sub)])
    pltpu.sync_copy(idx_hbm.at[pl.ds(chunk0 * 2 * CHUNK, 2 * ew)], idx_all)
    plsc.subcore_barrier()

    def fire(ch, b):
      iu = idx_all.at[pl.ds(ch * 2 * CHUNK, CHUNK)]
      iv = idx_all.at[pl.ds(ch * 2 * CHUNK + CHUNK, CHUNK)]
      pltpu.async_copy(h_sh.at[iu], rows_u[b], sem_u[b])
      pltpu.async_copy(h_sh.at[iv], rows_v[b], sem_v[b])

    def wait_gather(ch, b):
      iu = idx_all.at[pl.ds(ch * 2 * CHUNK, CHUNK)]
      iv = idx_all.at[pl.ds(ch * 2 * CHUNK + CHUNK, CHUNK)]
      pltpu.make_async_copy(h_sh.at[iu], rows_u[b], sem_u[b]).wait()
      pltpu.make_async_copy(h_sh.at[iv], rows_v[b], sem_v[b]).wait()

    for b in range(NBUF):
      fire(b, b)

    def loop_body(j, carry):
      for b in range(NBUF):
        ch = NBUF * j + b
        wait_gather(ch, b)

        def group_body(g, carry2, b=b):
          acc = jnp.zeros((LANES,), jnp.float32)
          for k in range(LANES):
            e = g * LANES + k
            ps = []
            for i in range(WPF):
              uw = plsc.bitcast(rows_u[b][e, pl.ds(i * LANES, LANES)],
                                jnp.bfloat16)
              vw = plsc.bitcast(rows_v[b][e, pl.ds(i * LANES, LANES)],
                                jnp.bfloat16)
              lo, hi = plsc.unpack(uw * vw, format=plsc.PackFormat.INTERLEAVED)
              ps.append(lo + hi)
            m = (ps[0] + ps[1]) + (ps[2] + ps[3])
            for p in rots:
              m = m + rot(m, p)
            t = m if k == 0 else rot(m, places[k])
            acc = jnp.where(lane == k, t, acc)
          out_all[pl.ds(ch * CHUNK + g * LANES, LANES)] = acc
          return carry2

        lax.fori_loop(0, GROUPS, group_body, 0)
        fire(jnp.minimum(ch + NBUF, n_chunks - 1), b)
      return carry

    lax.fori_loop(0, n_chunks // NBUF, loop_body, 0)
    for b in range(NBUF):
      wait_gather(0, b)
    pltpu.sync_copy(out_all, out_hbm.at[pl.ds(base, ew)])

  return score_kernel


def kernel(h, edge_index):
  e = edge_index.shape[1]
  epc = N_WORKERS * CHUNK * NBUF
  e_pad = ((e + epc - 1) // epc) * epc
  src = edge_index[0].astype(jnp.int32)
  dst = edge_index[1].astype(jnp.int32)
  if e_pad != e:
    src = jnp.pad(src, (0, e_pad - e))
    dst = jnp.pad(dst, (0, e_pad - e))
  idx = jnp.stack([src.reshape(-1, CHUNK), dst.reshape(-1, CHUNK)],
                  axis=1).reshape(-1)
  npc = 8 * N_SUBCORES
  n_pad = ((h.shape[0] + npc - 1) // npc) * npc
  if n_pad != h.shape[0]:
    h = jnp.pad(h, ((0, n_pad - h.shape[0]), (0, 0)))
  hw = jax.lax.bitcast_convert_type(
      h.astype(jnp.bfloat16).reshape(n_pad, D_FEAT // 2, 2), jnp.float32)
  out = _make_kernel(e_pad, n_pad)(hw, idx)
  return out[:e, None]


# speedup vs baseline: 6.6143x; 1.0016x over previous
import functools

import jax
import jax.numpy as jnp
from jax import lax
from jax.experimental import pallas as pl
from jax.experimental.pallas import tpu as pltpu
from jax.experimental.pallas import tpu_sc as plsc

D_FEAT = 128
LANES = 16
N_CORES = 2
N_SUBCORES = 16
N_WORKERS = N_CORES * N_SUBCORES
CHUNK = 128
GROUPS = CHUNK // LANES
D_WORDS = D_FEAT // 2
WPF = D_WORDS // LANES
NBUF = 2


def _make_kernel(e_pad, n_pad):
  ew = e_pad // N_WORKERS
  n_chunks = ew // CHUNK
  assert n_chunks % NBUF == 0
  assert n_pad % (8 * N_SUBCORES) == 0
  rows_per_sub = n_pad // N_SUBCORES
  mesh = plsc.VectorSubcoreMesh(core_axis_name="c", subcore_axis_name="s")

  @functools.partial(
      pl.kernel,
      mesh=mesh,
      compiler_params=pltpu.CompilerParams(needs_layout_passes=False,
                                           use_tc_tiling_on_sc=False),
      out_type=jax.ShapeDtypeStruct((e_pad,), jnp.float32),
      scratch_types=[
          pltpu.VMEM_SHARED((n_pad, D_WORDS), jnp.float32),
          pltpu.VMEM((2 * ew,), jnp.int32),
          pltpu.VMEM((ew,), jnp.float32),
      ] + [pltpu.VMEM((2 * CHUNK, D_WORDS), jnp.float32)] * NBUF
        + [pltpu.SemaphoreType.DMA] * NBUF,
  )
  def score_kernel(h_hbm, idx_hbm, out_hbm, h_sh, idx_all, out_all, *rest):
    rows = rest[:NBUF]
    sems = rest[NBUF:]

    cid = lax.axis_index("c")
    sid = lax.axis_index("s")
    wid = sid * N_CORES + cid
    base = wid * ew
    chunk0 = wid * n_chunks
    lane = lax.iota(jnp.int32, LANES)
    rots = [jnp.bitwise_and(lane + r, LANES - 1) for r in (8, 4, 2, 1)]
    places = [jnp.bitwise_and(lane - k, LANES - 1) for k in range(LANES)]

    def rot(x, perm):
      return x.at[perm].get(mode="promise_in_bounds")

    pltpu.sync_copy(h_hbm.at[pl.ds(sid * rows_per_sub, rows_per_sub)],
                    h_sh.at[pl.ds(sid * rows_per_sub, rows_per_sub)])
    pltpu.sync_copy(idx_hbm.at[pl.ds(chunk0 * 2 * CHUNK, 2 * ew)], idx_all)
    plsc.subcore_barrier()

    def fire(ch, b):
      ii = idx_all.at[pl.ds(ch * 2 * CHUNK, 2 * CHUNK)]
      pltpu.async_copy(h_sh.at[ii], rows[b], sems[b])

    def wait_gather(ch, b):
      ii = idx_all.at[pl.ds(ch * 2 * CHUNK, 2 * CHUNK)]
      pltpu.make_async_copy(h_sh.at[ii], rows[b], sems[b]).wait()

    for b in range(NBUF):
      fire(b, b)

    def loop_body(j, carry):
      for b in range(NBUF):
        ch = NBUF * j + b
        wait_gather(ch, b)

        def group_body(g, carry2, b=b):
          acc = jnp.zeros((LANES,), jnp.float32)
          for k in range(LANES):
            e = g * LANES + k
            ps = []
            for i in range(WPF):
              uw = plsc.bitcast(rows[b][e, pl.ds(i * LANES, LANES)],
                                jnp.bfloat16)
              vw = plsc.bitcast(rows[b][CHUNK + e, pl.ds(i * LANES, LANES)],
                                jnp.bfloat16)
              lo, hi = plsc.unpack(uw * vw, format=plsc.PackFormat.INTERLEAVED)
              ps.append(lo + hi)
            m = (ps[0] + ps[1]) + (ps[2] + ps[3])
            for p in rots:
              m = m + rot(m, p)
            t = m if k == 0 else rot(m, places[k])
            acc = jnp.where(lane == k, t, acc)
          out_all[pl.ds(ch * CHUNK + g * LANES, LANES)] = acc
          return carry2

        lax.fori_loop(0, GROUPS, group_body, 0)
        fire(jnp.minimum(ch + NBUF, n_chunks - 1), b)
      return carry

    lax.fori_loop(0, n_chunks // NBUF, loop_body, 0)
    for b in range(NBUF):
      wait_gather(0, b)
    pltpu.sync_copy(out_all, out_hbm.at[pl.ds(base, ew)])

  return score_kernel


def kernel(h, edge_index):
  e = edge_index.shape[1]
  epc = N_WORKERS * CHUNK * NBUF
  e_pad = ((e + epc - 1) // epc) * epc
  src = edge_index[0].astype(jnp.int32)
  dst = edge_index[1].astype(jnp.int32)
  if e_pad != e:
    src = jnp.pad(src, (0, e_pad - e))
    dst = jnp.pad(dst, (0, e_pad - e))
  idx = jnp.stack([src.reshape(-1, CHUNK), dst.reshape(-1, CHUNK)],
                  axis=1).reshape(-1)
  npc = 8 * N_SUBCORES
  n_pad = ((h.shape[0] + npc - 1) // npc) * npc
  if n_pad != h.shape[0]:
    h = jnp.pad(h, ((0, n_pad - h.shape[0]), (0, 0)))
  hw = jax.lax.bitcast_convert_type(
      h.astype(jnp.bfloat16).reshape(n_pad, D_FEAT // 2, 2), jnp.float32)
  out = _make_kernel(e_pad, n_pad)(hw, idx)
  return out[:e, None]
